# Initial kernel scaffold; baseline (speedup 1.0000x reference)
import jax
import jax.numpy as jnp
from jax.experimental import pallas as pl


def kernel(user_indices, pos_item_indices, neg_item_indices, user_table, item_table, W1_0, b1_0, W1_1, b1_1, W1_2, b1_2):
    raise NotImplementedError("write your pallas kernel here")


# speedup vs baseline: 2.3502x; 2.3502x over previous
import functools

import jax
import jax.numpy as jnp
from jax import lax
from jax.experimental import pallas as pl
from jax.experimental.pallas import tpu as pltpu
from jax.experimental.pallas import tpu_sc as plsc

NC, NS = 2, 16
NW = NC * NS
B = 16384
D = 64
CHUNK = 128
ROWS_PER_W = B // NW
NCHUNK = ROWS_PER_W // CHUNK
IDX_ROWS = B // CHUNK

BLK = 2048


def _lrelu(x):
    return jnp.where(x >= 0, x, 0.3 * x)


@functools.cache
def _make_sc_gather():
    mesh = plsc.VectorSubcoreMesh(
        core_axis_name="c", subcore_axis_name="s", num_cores=NC, num_subcores=NS
    )

    @functools.partial(
        pl.kernel,
        mesh=mesh,
        out_type=(
            jax.ShapeDtypeStruct((B, D), jnp.float32),
            jax.ShapeDtypeStruct((B, D), jnp.float32),
            jax.ShapeDtypeStruct((B, D), jnp.float32),
        ),
        scratch_types=(
            pltpu.VMEM((NCHUNK, CHUNK), jnp.int32),
            pltpu.VMEM((CHUNK, D), jnp.float32),
            pltpu.SemaphoreType.DMA,
        ),
        compiler_params=pltpu.CompilerParams(use_tc_tiling_on_sc=False),
    )
    def _sc_gather(user_tab, item_tab, uidx, pidx, nidx, u_out, p_out, n_out,
                   idx_v, rows_v, sem):
        wid = lax.axis_index("s") * NC + lax.axis_index("c")
        row0 = wid * NCHUNK
        for tab, idx, out in ((user_tab, uidx, u_out),
                              (item_tab, pidx, p_out),
                              (item_tab, nidx, n_out)):
            pltpu.sync_copy(idx.at[pl.ds(row0, NCHUNK)], idx_v)
            for c in range(NCHUNK):
                pltpu.async_copy(tab.at[idx_v.at[c]], rows_v, sem).wait()
                pltpu.sync_copy(rows_v, out.at[pl.ds((row0 + c) * CHUNK, CHUNK)])

    return _sc_gather


def _tc_body(u_ref, p_ref, n_ref, w0_ref, b0_ref, w1_ref, b1_ref,
             w2_ref, b2_ref, o_ref):
    u = u_ref[...]
    p = p_ref[...]
    n = n_ref[...]
    acc = jnp.sum(u * (p - n), axis=1, keepdims=True)
    for w_ref, b_ref in ((w0_ref, b0_ref), (w1_ref, b1_ref), (w2_ref, b2_ref)):
        w = w_ref[...]
        b = b_ref[...]
        u = _lrelu(jnp.dot(u, w, preferred_element_type=jnp.float32) + b)
        p = _lrelu(jnp.dot(p, w, preferred_element_type=jnp.float32) + b)
        n = _lrelu(jnp.dot(n, w, preferred_element_type=jnp.float32) + b)
        acc = acc + jnp.sum(u * (p - n), axis=1, keepdims=True)
    o_ref[...] = acc


_tc_score = pl.pallas_call(
    _tc_body,
    grid=(B // BLK,),
    in_specs=[
        pl.BlockSpec((BLK, D), lambda i: (i, 0)),
        pl.BlockSpec((BLK, D), lambda i: (i, 0)),
        pl.BlockSpec((BLK, D), lambda i: (i, 0)),
        pl.BlockSpec((D, D), lambda i: (0, 0)),
        pl.BlockSpec((1, D), lambda i: (0, 0)),
        pl.BlockSpec((D, D), lambda i: (0, 0)),
        pl.BlockSpec((1, D), lambda i: (0, 0)),
        pl.BlockSpec((D, D), lambda i: (0, 0)),
        pl.BlockSpec((1, D), lambda i: (0, 0)),
    ],
    out_specs=pl.BlockSpec((BLK, 1), lambda i: (i, 0)),
    out_shape=jax.ShapeDtypeStruct((B, 1), jnp.float32),
)


def kernel(user_indices, pos_item_indices, neg_item_indices, user_table,
           item_table, W1_0, b1_0, W1_1, b1_1, W1_2, b1_2):
    ui = user_indices.astype(jnp.int32).reshape(IDX_ROWS, CHUNK)
    pi = pos_item_indices.astype(jnp.int32).reshape(IDX_ROWS, CHUNK)
    ni = neg_item_indices.astype(jnp.int32).reshape(IDX_ROWS, CHUNK)
    u, p, n = _make_sc_gather()(user_table, item_table, ui, pi, ni)
    return _tc_score(u, p, n,
                     W1_0, b1_0.reshape(1, D),
                     W1_1, b1_1.reshape(1, D),
                     W1_2, b1_2.reshape(1, D))
